# baseline (speedup 1.0000x reference)
import functools

import jax
import jax.numpy as jnp
from jax import lax
from jax.experimental import pallas as pl
from jax.experimental.pallas import tpu as pltpu
from jax.experimental.pallas import tpu_sc as plsc

N = 10000
E = 320000
D = 128
DG = 16

NC = 2
NS = 16
NW = NC * NS

NP = 10240
RPT = NP // NS
EW = E // NW

_mesh = plsc.VectorSubcoreMesh(core_axis_name="c", subcore_axis_name="s")


def _make_seg_body(W, CH, NBUF, LOOK, R):
    NCH = EW // CH

    def body(tab_hbm, src_hbm, dst_hbm, zeros_hbm, out0_hbm, out1_hbm,
             src_v, dst_v, rows_v, acc_sh, *sems):
        gs, ss = sems[:NBUF], sems[NBUF:]
        cid = lax.axis_index("c")
        sid = lax.axis_index("s")
        wid = cid * NS + sid

        d0 = pltpu.async_copy(zeros_hbm.at[pl.ds(sid * RPT, RPT)],
                              acc_sh.at[pl.ds(sid * RPT, RPT)], sems[0])
        d1 = pltpu.async_copy(src_hbm.at[pl.ds(wid * NCH, NCH)], src_v,
                              sems[1])
        d2 = pltpu.async_copy(dst_hbm.at[pl.ds(wid * NCH, NCH)], dst_v,
                              sems[2])
        d0.wait()
        d1.wait()
        d2.wait()
        plsc.subcore_barrier()

        def round_body(it, carry):
            base = it * R

            def gather(s):
                b = s % NBUF
                return pltpu.async_copy(tab_hbm.at[src_v.at[base + s]],
                                        rows_v.at[b], gs[b])

            def scatter(s):
                b = s % NBUF
                return pltpu.async_copy(rows_v.at[b],
                                        acc_sh.at[dst_v.at[base + s]],
                                        ss[b], add=True)

            gd = [gather(s) for s in range(LOOK)] + [None] * (R - LOOK)
            sd = [None] * R
            for s in range(R):
                if s >= NBUF - LOOK:
                    sd[s - (NBUF - LOOK)].wait()
                if s + LOOK < R:
                    gd[s + LOOK] = gather(s + LOOK)
                gd[s].wait()
                sd[s] = scatter(s)
            for s in range(R - (NBUF - LOOK), R):
                sd[s].wait()
            return carry

        lax.fori_loop(0, NCH // R, round_body, 0, unroll=False)
        plsc.subcore_barrier()

        @pl.when(cid == 0)
        def _():
            pltpu.sync_copy(acc_sh.at[pl.ds(sid * RPT, RPT)],
                            out0_hbm.at[pl.ds(sid * RPT, RPT)])

        @pl.when(cid == 1)
        def _():
            pltpu.sync_copy(acc_sh.at[pl.ds(sid * RPT, RPT)],
                            out1_hbm.at[pl.ds(sid * RPT, RPT)])

    return body


def _make_seg_kernel(W, CH, NBUF, LOOK, R):
    NCH = EW // CH
    return functools.partial(
        pl.kernel,
        out_type=[jax.ShapeDtypeStruct((NP, W), jnp.float32),
                  jax.ShapeDtypeStruct((NP, W), jnp.float32)],
        mesh=_mesh,
        compiler_params=pltpu.CompilerParams(use_tc_tiling_on_sc=False),
        scratch_types=[
            pltpu.VMEM((NCH, CH), jnp.int32),
            pltpu.VMEM((NCH, CH), jnp.int32),
            pltpu.VMEM((NBUF, CH, W), jnp.float32),
            pltpu.VMEM_SHARED((NP, W), jnp.float32),
        ] + [pltpu.SemaphoreType.DMA] * (2 * NBUF),
    )(_make_seg_body(W, CH, NBUF, LOOK, R))


CH1 = 40

_seg_rows = _make_seg_kernel(D, CH1, 4, 2, 50)


BLK = 2000


def _mid_kernel(a0_ref, a1_ref, x_ref, wrel1_ref, brel1_ref, wroot1_ref,
                h_ref):
    a = a0_ref[...] + a1_ref[...]
    h = jnp.dot(a, wrel1_ref[...].T, preferred_element_type=jnp.float32)
    h = h + jnp.dot(x_ref[...], wroot1_ref[...].T,
                    preferred_element_type=jnp.float32)
    h_ref[...] = jnp.maximum(h + brel1_ref[...], 0.0)


def _mid_stage(s0, s1, x, Wrel1, brel1, Wroot1):
    full = pl.BlockSpec((D, D), lambda i: (0, 0))
    row1 = pl.BlockSpec((1, D), lambda i: (0, 0))
    blk = pl.BlockSpec((BLK, D), lambda i: (i, 0))
    return pl.pallas_call(
        _mid_kernel,
        grid=(N // BLK,),
        in_specs=[blk, blk, blk, full, row1, full],
        out_specs=blk,
        out_shape=jax.ShapeDtypeStruct((NP, D), jnp.float32),
    )(s0, s1, x, Wrel1, brel1.reshape(1, D), Wroot1)


def _final_kernel(a0_ref, a1_ref, h_ref, wrel2_ref, brel2_ref, wroot2_ref,
                  wfcT_ref, bfc_ref, o_ref):
    a = a0_ref[...] + a1_ref[...]
    h2 = jnp.dot(a, wrel2_ref[...].T, preferred_element_type=jnp.float32)
    h2 = h2 + jnp.dot(h_ref[...], wroot2_ref[...].T,
                      preferred_element_type=jnp.float32)
    h2 = h2 + brel2_ref[...]
    o = jnp.dot(h2, wfcT_ref[...], preferred_element_type=jnp.float32)
    o_ref[...] = jnp.sum(o, axis=1) + bfc_ref[0, 0]


BLK4 = 2048


def _final_stage(s20, s21, h, Wrel2, brel2, Wroot2, Wfc, bfc):
    full = pl.BlockSpec((D, D), lambda i: (0, 0))
    row1 = pl.BlockSpec((1, D), lambda i: (0, 0))
    blk4 = pl.BlockSpec((BLK4, D), lambda i: (i, 0))
    out = pl.pallas_call(
        _final_kernel,
        grid=(NP // BLK4,),
        in_specs=[blk4, blk4, blk4, full, row1, full,
                  pl.BlockSpec((D, 1), lambda i: (0, 0)),
                  pl.BlockSpec((1, 1), lambda i: (0, 0))],
        out_specs=pl.BlockSpec((BLK4,), lambda i: (i,)),
        out_shape=jax.ShapeDtypeStruct((NP,), jnp.float32),
    )(s20, s21, h, Wrel2, brel2.reshape(1, D), Wroot2,
      Wfc.reshape(D, 1), bfc.reshape(1, 1))
    return out[:N]


def kernel(x, edge_index, Wrel1, brel1, Wroot1, Wrel2, brel2, Wroot2, Wfc, bfc):
    src = edge_index[0].astype(jnp.int32)
    dst = edge_index[1].astype(jnp.int32)
    src1 = src.reshape(E // CH1, CH1)
    dst1 = dst.reshape(E // CH1, CH1)
    zeros = jnp.zeros((NP, D), jnp.float32)

    s0, s1 = _seg_rows(x, src1, dst1, zeros)
    h = _mid_stage(s0, s1, x, Wrel1, brel1, Wroot1)
    s20, s21 = _seg_rows(h, src1, dst1, zeros)
    return _final_stage(s20, s21, h, Wrel2, brel2, Wroot2, Wfc, bfc)


# speedup vs baseline: 1.0245x; 1.0245x over previous
import functools

import jax
import jax.numpy as jnp
from jax import lax
from jax.experimental import pallas as pl
from jax.experimental.pallas import tpu as pltpu
from jax.experimental.pallas import tpu_sc as plsc

N = 10000
E = 320000
D = 128
DG = 16

NC = 2
NS = 16
NW = NC * NS

NP = 10240
RPT = NP // NS
EW = E // NW

_mesh = plsc.VectorSubcoreMesh(core_axis_name="c", subcore_axis_name="s")


def _make_seg_body(W, CH, NBUF, LOOK, R):
    NCH = EW // CH

    def body(tab_hbm, src_hbm, dst_hbm, zeros_hbm, out0_hbm, out1_hbm,
             src_v, dst_v, rows_v, acc_sh, *sems):
        gs, ss = sems[:NBUF], sems[NBUF:]
        cid = lax.axis_index("c")
        sid = lax.axis_index("s")
        wid = cid * NS + sid

        d0 = pltpu.async_copy(zeros_hbm.at[pl.ds(sid * RPT, RPT)],
                              acc_sh.at[pl.ds(sid * RPT, RPT)], sems[0])
        d1 = pltpu.async_copy(src_hbm.at[pl.ds(wid * NCH, NCH)], src_v,
                              sems[1])
        d2 = pltpu.async_copy(dst_hbm.at[pl.ds(wid * NCH, NCH)], dst_v,
                              sems[2])
        d0.wait()
        d1.wait()
        d2.wait()
        plsc.subcore_barrier()

        def round_body(it, carry):
            base = it * R

            def gather(s):
                b = s % NBUF
                return pltpu.async_copy(tab_hbm.at[src_v.at[base + s]],
                                        rows_v.at[b], gs[b])

            def scatter(s):
                b = s % NBUF
                return pltpu.async_copy(rows_v.at[b],
                                        acc_sh.at[dst_v.at[base + s]],
                                        ss[b], add=True)

            gd = [gather(s) for s in range(LOOK)] + [None] * (R - LOOK)
            sd = [None] * R
            for s in range(R):
                if s >= NBUF - LOOK:
                    sd[s - (NBUF - LOOK)].wait()
                if s + LOOK < R:
                    gd[s + LOOK] = gather(s + LOOK)
                gd[s].wait()
                sd[s] = scatter(s)
            for s in range(R - (NBUF - LOOK), R):
                sd[s].wait()
            return carry

        lax.fori_loop(0, NCH // R, round_body, 0, unroll=False)
        plsc.subcore_barrier()

        @pl.when(cid == 0)
        def _():
            pltpu.sync_copy(acc_sh.at[pl.ds(sid * RPT, RPT)],
                            out0_hbm.at[pl.ds(sid * RPT, RPT)])

        @pl.when(cid == 1)
        def _():
            pltpu.sync_copy(acc_sh.at[pl.ds(sid * RPT, RPT)],
                            out1_hbm.at[pl.ds(sid * RPT, RPT)])

    return body


def _make_seg_kernel(W, CH, NBUF, LOOK, R):
    NCH = EW // CH
    return functools.partial(
        pl.kernel,
        out_type=[jax.ShapeDtypeStruct((NP, W), jnp.float32),
                  jax.ShapeDtypeStruct((NP, W), jnp.float32)],
        mesh=_mesh,
        compiler_params=pltpu.CompilerParams(use_tc_tiling_on_sc=False),
        scratch_types=[
            pltpu.VMEM((NCH, CH), jnp.int32),
            pltpu.VMEM((NCH, CH), jnp.int32),
            pltpu.VMEM((NBUF, CH, W), jnp.float32),
            pltpu.VMEM_SHARED((NP, W), jnp.float32),
        ] + [pltpu.SemaphoreType.DMA] * (2 * NBUF),
    )(_make_seg_body(W, CH, NBUF, LOOK, R))


CH1 = 40

_seg_rows = _make_seg_kernel(D, CH1, 4, 2, 125)


BLK = 2000


def _mid_kernel(a0_ref, a1_ref, x_ref, wrel1_ref, brel1_ref, wroot1_ref,
                h_ref):
    a = a0_ref[...] + a1_ref[...]
    h = jnp.dot(a, wrel1_ref[...].T, preferred_element_type=jnp.float32)
    h = h + jnp.dot(x_ref[...], wroot1_ref[...].T,
                    preferred_element_type=jnp.float32)
    h_ref[...] = jnp.maximum(h + brel1_ref[...], 0.0)


def _mid_stage(s0, s1, x, Wrel1, brel1, Wroot1):
    full = pl.BlockSpec((D, D), lambda i: (0, 0))
    row1 = pl.BlockSpec((1, D), lambda i: (0, 0))
    blk = pl.BlockSpec((BLK, D), lambda i: (i, 0))
    return pl.pallas_call(
        _mid_kernel,
        grid=(N // BLK,),
        in_specs=[blk, blk, blk, full, row1, full],
        out_specs=blk,
        out_shape=jax.ShapeDtypeStruct((NP, D), jnp.float32),
    )(s0, s1, x, Wrel1, brel1.reshape(1, D), Wroot1)


def _final_kernel(a0_ref, a1_ref, h_ref, wrel2_ref, brel2_ref, wroot2_ref,
                  wfcT_ref, bfc_ref, o_ref):
    a = a0_ref[...] + a1_ref[...]
    h2 = jnp.dot(a, wrel2_ref[...].T, preferred_element_type=jnp.float32)
    h2 = h2 + jnp.dot(h_ref[...], wroot2_ref[...].T,
                      preferred_element_type=jnp.float32)
    h2 = h2 + brel2_ref[...]
    o = jnp.dot(h2, wfcT_ref[...], preferred_element_type=jnp.float32)
    o_ref[...] = jnp.sum(o, axis=1) + bfc_ref[0, 0]


BLK4 = 2048


def _final_stage(s20, s21, h, Wrel2, brel2, Wroot2, Wfc, bfc):
    full = pl.BlockSpec((D, D), lambda i: (0, 0))
    row1 = pl.BlockSpec((1, D), lambda i: (0, 0))
    blk4 = pl.BlockSpec((BLK4, D), lambda i: (i, 0))
    out = pl.pallas_call(
        _final_kernel,
        grid=(NP // BLK4,),
        in_specs=[blk4, blk4, blk4, full, row1, full,
                  pl.BlockSpec((D, 1), lambda i: (0, 0)),
                  pl.BlockSpec((1, 1), lambda i: (0, 0))],
        out_specs=pl.BlockSpec((BLK4,), lambda i: (i,)),
        out_shape=jax.ShapeDtypeStruct((NP,), jnp.float32),
    )(s20, s21, h, Wrel2, brel2.reshape(1, D), Wroot2,
      Wfc.reshape(D, 1), bfc.reshape(1, 1))
    return out[:N]


def kernel(x, edge_index, Wrel1, brel1, Wroot1, Wrel2, brel2, Wroot2, Wfc, bfc):
    src = edge_index[0].astype(jnp.int32)
    dst = edge_index[1].astype(jnp.int32)
    src1 = src.reshape(E // CH1, CH1)
    dst1 = dst.reshape(E // CH1, CH1)
    zeros = jnp.zeros((NP, D), jnp.float32)

    s0, s1 = _seg_rows(x, src1, dst1, zeros)
    h = _mid_stage(s0, s1, x, Wrel1, brel1, Wroot1)
    s20, s21 = _seg_rows(h, src1, dst1, zeros)
    return _final_stage(s20, s21, h, Wrel2, brel2, Wroot2, Wfc, bfc)
